# baseline (speedup 1.0000x reference)
import functools

import jax
import jax.numpy as jnp
from jax import lax
from jax.experimental import pallas as pl
from jax.experimental.pallas import tpu as pltpu
from jax.experimental.pallas import tpu_sc as plsc

N = 100000
NTILES = 32
RANGE = N // NTILES
NBINS = 3136
CHUNK = 4000
NCHUNKS = N // CHUNK
VPC = CHUNK // 16
TRASH = 16
C = 80

_mesh = plsc.VectorSubcoreMesh(core_axis_name="c", subcore_axis_name="s")
_params = pltpu.CompilerParams(needs_layout_passes=False)


@functools.partial(
    pl.kernel,
    out_type=jax.ShapeDtypeStruct((N + TRASH,), jnp.int32),
    mesh=_mesh,
    scratch_types=[
        pltpu.VMEM((CHUNK,), jnp.int32),
        pltpu.VMEM((NBINS,), jnp.int32),
        pltpu.VMEM((800, 128), jnp.int32),
        pltpu.VMEM((128,), jnp.int32),
    ],
    compiler_params=_params,
)
def _rank_kernel(v_hbm, rank_hbm, vstage, cnt, ord2, vbuf):
    wid = lax.axis_index("s") * 2 + lax.axis_index("c")
    lo = wid * RANGE
    hi = lo + RANGE
    iota = lax.iota(jnp.int32, 16)

    cc, _ = plsc.scan_count(jnp.zeros((16,), jnp.int32))
    c0 = jnp.min(cc)

    def zero_body(i, _):
        cnt[pl.ds(i * 16, 16)] = jnp.zeros((16,), jnp.int32)
        return 0
    lax.fori_loop(0, NBINS // 16, zero_body, 0)

    def scan_a_chunk(p, nbv):
        pltpu.sync_copy(v_hbm.at[pl.ds(p * CHUNK, CHUNK)], vstage)

        def body(i, nbv):
            x = vstage[pl.ds(i * 16, 16)]
            m = (x >= lo) & (x < hi)
            cntv, last = plsc.scan_count(x, m)
            plsc.addupdate_scatter(cnt, [x - lo], cntv - c0 + 1, mask=last)
            nbv = nbv + plsc.all_reduce_population_count(x < lo)
            return nbv
        return lax.fori_loop(0, VPC, body, nbv)

    nbv = lax.fori_loop(0, NCHUNKS, scan_a_chunk, jnp.zeros((16,), jnp.int32))
    base = jnp.max(nbv)

    def scan_body(i, s0):
        c = cnt[pl.ds(i * 16, 16)]
        cs = plsc.cumsum(c)
        cnt[pl.ds(i * 16, 16)] = cs - c + s0
        return s0 + jnp.max(cs)
    total_end = lax.fori_loop(0, NBINS // 16, scan_body, base)
    t_cnt = total_end - base

    def scan_b_chunk(p, _):
        pltpu.sync_copy(v_hbm.at[pl.ds(p * CHUNK, CHUNK)], vstage)

        def body(i, _):
            x = vstage[pl.ds(i * 16, 16)]
            m = (x >= lo) & (x < hi)
            b = x - lo
            pos = plsc.load_gather(cnt, [b], mask=m)
            cntv, last = plsc.scan_count(x, m)
            pl_pos = pos + (cntv - c0) - base
            jvec = (p * CHUNK + i * 16) + iota
            plsc.store_scatter(
                ord2, [lax.shift_right_logical(pl_pos, 7), pl_pos & 127],
                jvec, mask=m)
            plsc.addupdate_scatter(cnt, [b], cntv - c0 + 1, mask=last)
            return 0
        lax.fori_loop(0, VPC, body, 0)
        return 0
    lax.fori_loop(0, NCHUNKS, scan_b_chunk, 0)

    nch = (t_cnt + 127) // 128

    @pl.when(t_cnt > 0)
    def _():
        lr = nch - 1
        for mloc in range(8):
            lane_pos = lr * 128 + mloc * 16 + iota
            valid = lane_pos < t_cnt
            r = ord2[lr, pl.ds(mloc * 16, 16)]
            ord2[lr, pl.ds(mloc * 16, 16)] = jnp.where(valid, r, N + iota)

    def inv_body(k, _):
        for mloc in range(8):
            vbuf[pl.ds(mloc * 16, 16)] = (base + k * 128 + mloc * 16) + iota
        pltpu.sync_copy(vbuf, rank_hbm.at[ord2.at[k]])
        return 0
    lax.fori_loop(0, nch, inv_body, 0)


@functools.partial(
    pl.kernel,
    out_type=jax.ShapeDtypeStruct((N, 256), jnp.float32),
    mesh=_mesh,
    scratch_types=[
        pltpu.VMEM((1, C), jnp.int32),
        pltpu.VMEM((C, 256), jnp.float32),
    ],
    compiler_params=_params,
)
def _scatter_kernel(asc_hbm, cru_hbm, des_hbm, rank_hbm, out_hbm, ibuf, dbuf):
    wid = lax.axis_index("s") * 2 + lax.axis_index("c")

    def run_source(src_ref, offset, nchunks, kmax):
        def body(k, _):
            c = wid + NTILES * k

            @pl.when(c < nchunks)
            def _():
                pltpu.sync_copy(
                    rank_hbm.at[pl.ds(offset + c * C, C)], ibuf.at[0])
                pltpu.sync_copy(src_ref.at[pl.ds(c * C, C), :], dbuf)
                pltpu.sync_copy(dbuf, out_hbm.at[ibuf.at[0]])
            return 0
        lax.fori_loop(0, kmax, body, 0)

    run_source(asc_hbm, 0, 40000 // C, 16)
    run_source(cru_hbm, 40000, 40000 // C, 16)
    run_source(des_hbm, 80000, 20000 // C, 8)


def kernel(asc_dec, cru_dec, des_dec, concat_index):
    rank = _rank_kernel(concat_index.astype(jnp.int32))
    return _scatter_kernel(asc_dec, cru_dec, des_dec, rank)
